# initial kernel scaffold (unmeasured)
import jax
import jax.numpy as jnp
from jax import lax
from jax.experimental import pallas as pl
from jax.experimental.pallas import tpu as pltpu

N_DEV = 4
KC = 1024


def kernel(x, w_mat):
    M, K = x.shape
    _, N = w_mat.shape
    NB = N // N_DEV
    NK = K // KC

    my = lax.axis_index("i")
    offs = jnp.array([1, 2, 3, 0], dtype=jnp.int32)
    targets = (my + offs) % N_DEV

    def body(targ_ref, x_ref, w_ref, out_ref,
             acc_ref, send_bufs, recv_bufs,
             send_sems, recv_sems, copy_sems):
        t = pl.program_id(0)
        k = pl.program_id(1)
        my_pos = lax.axis_index("i")

        @pl.when((t == 0) & (k == 0))
        def _():
            barrier = pltpu.get_barrier_semaphore()
            for d in range(1, N_DEV):
                pl.semaphore_signal(
                    barrier, inc=1,
                    device_id=((my_pos + d) % N_DEV,),
                    device_id_type=pl.DeviceIdType.MESH,
                )
            pl.semaphore_wait(barrier, N_DEV - 1)

        prod = jnp.dot(x_ref[...], w_ref[...],
                       preferred_element_type=jnp.float32)

        @pl.when(k == 0)
        def _():
            acc_ref[...] = prod

        @pl.when(k > 0)
        def _():
            acc_ref[...] += prod

        for tt in range(N_DEV):
            @pl.when((k == NK - 1) & (t == tt))
            def _(tt=tt):
                y = jax.nn.gelu(acc_ref[...], approximate=True)
                send_bufs[tt] = y.astype(jnp.bfloat16)

                if tt < N_DEV - 1:
                    rdma = pltpu.make_async_remote_copy(
                        src_ref=send_bufs.at[tt],
                        dst_ref=recv_bufs.at[tt],
                        send_sem=send_sems.at[tt],
                        recv_sem=recv_sems.at[tt],
                        device_id=(targ_ref[tt],),
                        device_id_type=pl.DeviceIdType.MESH,
                    )
                    rdma.start()
                else:
                    own_copy = pltpu.make_async_copy(
                        send_bufs.at[tt],
                        out_ref.at[pl.ds(my_pos * M, M), :],
                        copy_sems.at[N_DEV - 1],
                    )
                    own_copy.start()

                    out_copies = [own_copy]
                    for r in range(N_DEV - 1):
                        src = (my_pos - (r + 1)) % N_DEV
                        recv_desc = pltpu.make_async_remote_copy(
                            src_ref=recv_bufs.at[r],
                            dst_ref=recv_bufs.at[r],
                            send_sem=send_sems.at[r],
                            recv_sem=recv_sems.at[r],
                            device_id=(my_pos,),
                            device_id_type=pl.DeviceIdType.MESH,
                        )
                        recv_desc.wait_recv()
                        cp = pltpu.make_async_copy(
                            recv_bufs.at[r],
                            out_ref.at[pl.ds(src * M, M), :],
                            copy_sems.at[r],
                        )
                        cp.start()
                        out_copies.append(cp)

                    for r in range(N_DEV - 1):
                        send_desc = pltpu.make_async_remote_copy(
                            src_ref=send_bufs.at[r],
                            dst_ref=recv_bufs.at[r],
                            send_sem=send_sems.at[r],
                            recv_sem=recv_sems.at[r],
                            device_id=(targ_ref[r],),
                            device_id_type=pl.DeviceIdType.MESH,
                        )
                        send_desc.wait_send()
                    for cp in out_copies:
                        cp.wait()

    grid_spec = pltpu.PrefetchScalarGridSpec(
        num_scalar_prefetch=1,
        grid=(N_DEV, NK),
        in_specs=[
            pl.BlockSpec((M, KC), lambda t, k, targ: (0, k)),
            pl.BlockSpec((KC, NB), lambda t, k, targ: (k, targ[t])),
        ],
        out_specs=pl.BlockSpec(memory_space=pltpu.MemorySpace.ANY),
        scratch_shapes=[
            pltpu.VMEM((M, NB), jnp.float32),
            pltpu.VMEM((N_DEV, M, NB), jnp.bfloat16),
            pltpu.VMEM((N_DEV - 1, M, NB), jnp.bfloat16),
            pltpu.SemaphoreType.DMA((N_DEV - 1,)),
            pltpu.SemaphoreType.DMA((N_DEV - 1,)),
            pltpu.SemaphoreType.DMA((N_DEV,)),
        ],
    )

    return pl.pallas_call(
        body,
        grid_spec=grid_spec,
        out_shape=jax.ShapeDtypeStruct((N_DEV * M, NB), jnp.bfloat16),
        compiler_params=pltpu.CompilerParams(
            dimension_semantics=("arbitrary", "arbitrary"),
            collective_id=0,
        ),
    )(targets, x, w_mat)


# baseline (device time: 259406 ns/iter reference)
import jax
import jax.numpy as jnp
from jax import lax
from jax.experimental import pallas as pl
from jax.experimental.pallas import tpu as pltpu

N_DEV = 4
KC = 512


def kernel(x, w_mat):
    M, K = x.shape
    _, N = w_mat.shape
    NB = N // N_DEV
    NK = K // KC

    my = lax.axis_index("i")
    offs = jnp.array([1, 2, 3, 0], dtype=jnp.int32)
    targets = (my + offs) % N_DEV

    def body(targ_ref, x_ref, w_ref, dummy_ref, out_ref,
             acc_ref, send_bufs, send_sems, recv_sems, copy_sem):
        del dummy_ref
        t = pl.program_id(0)
        k = pl.program_id(1)
        my_pos = lax.axis_index("i")

        @pl.when((t == 0) & (k == 0))
        def _():
            barrier = pltpu.get_barrier_semaphore()
            for d in range(1, N_DEV):
                pl.semaphore_signal(
                    barrier, inc=1,
                    device_id=((my_pos + d) % N_DEV,),
                    device_id_type=pl.DeviceIdType.MESH,
                )
            pl.semaphore_wait(barrier, N_DEV - 1)

        prod = jnp.dot(x_ref[...].astype(jnp.bfloat16),
                       w_ref[...].astype(jnp.bfloat16),
                       preferred_element_type=jnp.float32)

        @pl.when(k == 0)
        def _():
            acc_ref[...] = prod

        @pl.when(k > 0)
        def _():
            acc_ref[...] += prod

        def send_desc(r):
            return pltpu.make_async_remote_copy(
                src_ref=send_bufs.at[r],
                dst_ref=out_ref.at[pl.ds(my_pos * M, M), :],
                send_sem=send_sems.at[r],
                recv_sem=recv_sems.at[r],
                device_id=(targ_ref[r],),
                device_id_type=pl.DeviceIdType.MESH,
            )

        for tt in range(N_DEV):
            @pl.when((k == NK - 1) & (t == tt))
            def _(tt=tt):
                y = jax.nn.gelu(acc_ref[...], approximate=True)

                if tt < N_DEV - 1:
                    send_bufs[tt] = y.astype(jnp.bfloat16)
                    send_desc(tt).start()
                else:
                    send_desc(0).wait_send()
                    send_bufs[0] = y.astype(jnp.bfloat16)
                    own_copy = pltpu.make_async_copy(
                        send_bufs.at[0],
                        out_ref.at[pl.ds(my_pos * M, M), :],
                        copy_sem,
                    )
                    own_copy.start()

                    for r in range(N_DEV - 1):
                        src = (my_pos - (r + 1)) % N_DEV
                        recv_desc = pltpu.make_async_remote_copy(
                            src_ref=send_bufs.at[r],
                            dst_ref=out_ref.at[pl.ds(src * M, M), :],
                            send_sem=send_sems.at[r],
                            recv_sem=recv_sems.at[r],
                            device_id=(my_pos,),
                            device_id_type=pl.DeviceIdType.MESH,
                        )
                        recv_desc.wait_recv()

                    for r in range(1, N_DEV - 1):
                        send_desc(r).wait_send()
                    own_copy.wait()

    grid_spec = pltpu.PrefetchScalarGridSpec(
        num_scalar_prefetch=1,
        grid=(N_DEV, NK),
        in_specs=[
            pl.BlockSpec((M, KC), lambda t, k, targ: (0, k)),
            pl.BlockSpec((KC, NB), lambda t, k, targ: (k, targ[t])),
            pl.BlockSpec(memory_space=pl.ANY),
        ],
        out_specs=pl.BlockSpec(memory_space=pl.ANY),
        scratch_shapes=[
            pltpu.VMEM((M, NB), jnp.float32),
            pltpu.VMEM((N_DEV - 1, M, NB), jnp.bfloat16),
            pltpu.SemaphoreType.DMA((N_DEV - 1,)),
            pltpu.SemaphoreType.DMA((N_DEV - 1,)),
            pltpu.SemaphoreType.DMA,
        ],
    )

    dummy = pltpu.with_memory_space_constraint(
        jnp.zeros((N_DEV * M, NB), jnp.bfloat16), pltpu.MemorySpace.HBM)

    return pl.pallas_call(
        body,
        grid_spec=grid_spec,
        out_shape=jax.ShapeDtypeStruct((N_DEV * M, NB), jnp.bfloat16),
        input_output_aliases={3: 0},
        compiler_params=pltpu.CompilerParams(
            dimension_semantics=("arbitrary", "arbitrary"),
            collective_id=0,
            vmem_limit_bytes=60 * 1024 * 1024,
        ),
    )(targets, x, w_mat, dummy)


# device time: 223457 ns/iter; 1.1609x vs baseline; 1.1609x over previous
import jax
import jax.numpy as jnp
from jax import lax
from jax.experimental import pallas as pl
from jax.experimental.pallas import tpu as pltpu

N_DEV = 4
KC = 1024


def kernel(x, w_mat):
    M, K = x.shape
    _, N = w_mat.shape
    NB = N // N_DEV
    NK = K // KC

    my = lax.axis_index("i")
    offs = jnp.array([1, 2, 3, 0], dtype=jnp.int32)
    targets = (my + offs) % N_DEV

    def body(targ_ref, x_ref, w_ref, dummy_ref, out_ref,
             acc_ref, send_bufs, send_sems, recv_sems, copy_sem):
        del dummy_ref
        t = pl.program_id(0)
        k = pl.program_id(1)
        my_pos = lax.axis_index("i")

        @pl.when((t == 0) & (k == 0))
        def _():
            barrier = pltpu.get_barrier_semaphore()
            for d in range(1, N_DEV):
                pl.semaphore_signal(
                    barrier, inc=1,
                    device_id=((my_pos + d) % N_DEV,),
                    device_id_type=pl.DeviceIdType.MESH,
                )
            pl.semaphore_wait(barrier, N_DEV - 1)

        prod = jnp.dot(x_ref[...].astype(jnp.bfloat16),
                       w_ref[...].astype(jnp.bfloat16),
                       preferred_element_type=jnp.float32)

        @pl.when(k == 0)
        def _():
            acc_ref[...] = prod

        @pl.when(k > 0)
        def _():
            acc_ref[...] += prod

        def send_desc(r):
            return pltpu.make_async_remote_copy(
                src_ref=send_bufs.at[r],
                dst_ref=out_ref.at[pl.ds(my_pos * M, M), :],
                send_sem=send_sems.at[r],
                recv_sem=recv_sems.at[r],
                device_id=(targ_ref[r],),
                device_id_type=pl.DeviceIdType.MESH,
            )

        for tt in range(N_DEV):
            @pl.when((k == NK - 1) & (t == tt))
            def _(tt=tt):
                y = jax.nn.gelu(acc_ref[...], approximate=True)

                if tt < N_DEV - 1:
                    send_bufs[tt] = y.astype(jnp.bfloat16)
                    send_desc(tt).start()
                else:
                    send_desc(0).wait_send()
                    send_bufs[0] = y.astype(jnp.bfloat16)
                    own_copy = pltpu.make_async_copy(
                        send_bufs.at[0],
                        out_ref.at[pl.ds(my_pos * M, M), :],
                        copy_sem,
                    )
                    own_copy.start()

                    for r in range(N_DEV - 1):
                        src = (my_pos - (r + 1)) % N_DEV
                        recv_desc = pltpu.make_async_remote_copy(
                            src_ref=send_bufs.at[r],
                            dst_ref=out_ref.at[pl.ds(src * M, M), :],
                            send_sem=send_sems.at[r],
                            recv_sem=recv_sems.at[r],
                            device_id=(my_pos,),
                            device_id_type=pl.DeviceIdType.MESH,
                        )
                        recv_desc.wait_recv()

                    for r in range(1, N_DEV - 1):
                        send_desc(r).wait_send()
                    own_copy.wait()

    grid_spec = pltpu.PrefetchScalarGridSpec(
        num_scalar_prefetch=1,
        grid=(N_DEV, NK),
        in_specs=[
            pl.BlockSpec((M, KC), lambda t, k, targ: (0, k)),
            pl.BlockSpec((KC, NB), lambda t, k, targ: (k, targ[t])),
            pl.BlockSpec(memory_space=pl.ANY),
        ],
        out_specs=pl.BlockSpec(memory_space=pl.ANY),
        scratch_shapes=[
            pltpu.VMEM((M, NB), jnp.float32),
            pltpu.VMEM((N_DEV - 1, M, NB), jnp.bfloat16),
            pltpu.SemaphoreType.DMA((N_DEV - 1,)),
            pltpu.SemaphoreType.DMA((N_DEV - 1,)),
            pltpu.SemaphoreType.DMA,
        ],
    )

    dummy = pltpu.with_memory_space_constraint(
        jnp.zeros((N_DEV * M, NB), jnp.bfloat16), pltpu.MemorySpace.HBM)

    return pl.pallas_call(
        body,
        grid_spec=grid_spec,
        out_shape=jax.ShapeDtypeStruct((N_DEV * M, NB), jnp.bfloat16),
        input_output_aliases={3: 0},
        compiler_params=pltpu.CompilerParams(
            dimension_semantics=("arbitrary", "arbitrary"),
            collective_id=0,
            vmem_limit_bytes=63 * 1024 * 1024,
        ),
    )(targets, x, w_mat, dummy)
